# baseline (speedup 1.0000x reference)
import jax
import jax.numpy as jnp
from jax.experimental import pallas as pl
from jax.experimental.pallas import tpu as pltpu

EMBED_DIM = 128


def _lookup(idx_ref, table_ref, out_ref, sem):
    i = idx_ref[0]
    cp = pltpu.make_async_copy(table_ref.at[i], out_ref, sem)
    cp.start()
    cp.wait()


def kernel(task_id, embedding_weight):
    idx = task_id.reshape(-1)[:1].astype(jnp.int32)
    return pl.pallas_call(
        _lookup,
        in_specs=[
            pl.BlockSpec(memory_space=pltpu.MemorySpace.SMEM),
            pl.BlockSpec(memory_space=pltpu.MemorySpace.HBM),
        ],
        out_specs=pl.BlockSpec(memory_space=pltpu.MemorySpace.HBM),
        out_shape=jax.ShapeDtypeStruct((EMBED_DIM,), jnp.float32),
        scratch_shapes=[pltpu.SemaphoreType.DMA],
    )(idx, embedding_weight)
